# baseline (speedup 1.0000x reference)
import dataclasses
import functools

import jax
import jax.numpy as jnp
from jax import lax
from jax.experimental import pallas as pl
from jax.experimental.pallas import tpu as pltpu
from jax.experimental.pallas import tpu_sc as plsc

N = 10000
R = 16
E = 320000
S = 4096
D = 128

NC = 2
NS = 16
NW = NC * NS
EPW = E // NW
CH = 80
NCH = EPW // CH
NBINS = 160256
BPW = NBINS // NS
NPAD = 10240
RPS = NPAD // NS

_mesh = plsc.VectorSubcoreMesh(core_axis_name="c", subcore_axis_name="s")

_cp = pltpu.CompilerParams()
if "needs_layout_passes" in pltpu.CompilerParams.__dataclass_fields__:
    _cp = dataclasses.replace(_cp, needs_layout_passes=False)


def _wid():
    return lax.axis_index("s") * NC + lax.axis_index("c")


@jax.jit
def _sc_mask(sub_idx, m_logits):
    @functools.partial(
        pl.kernel,
        out_type=jax.ShapeDtypeStruct((E,), jnp.float32),
        mesh=_mesh,
        scratch_types=[
            pltpu.VMEM((S,), jnp.int32),
            pltpu.VMEM((S,), jnp.float32),
            pltpu.VMEM((EPW,), jnp.float32),
            pltpu.SemaphoreType.DMA,
        ],
        compiler_params=_cp,
    )
    def k(idx_hbm, logit_hbm, out_hbm, idx_v, logit_v, buf, sem):
        wid = _wid()
        base = wid * EPW
        pltpu.async_copy(idx_hbm, idx_v, sem).wait()
        pltpu.async_copy(logit_hbm, logit_v, sem).wait()

        ones = jnp.full((16,), 1.0, jnp.float32)

        @pl.loop(0, EPW // 16)
        def _(i):
            buf[pl.ds(i * 16, 16)] = ones

        @pl.loop(0, S // 16)
        def _(j):
            iv = idx_v[pl.ds(j * 16, 16)]
            lv = logit_v[pl.ds(j * 16, 16)]
            val = 1.0 / (1.0 + jnp.exp(-lv))
            m = (iv >= base) & (iv < base + EPW)
            il = jnp.minimum(jnp.maximum(iv - base, 0), EPW - 1)
            plsc.store_scatter(buf, [il], val, mask=m)

        pltpu.async_copy(buf, out_hbm.at[pl.ds(base, EPW)], sem).wait()

    return k(sub_idx, m_logits)


@jax.jit
def _sc_hist(dst, rel):
    @functools.partial(
        pl.kernel,
        out_type=jax.ShapeDtypeStruct((NC * NBINS,), jnp.float32),
        mesh=_mesh,
        scratch_types=[
            pltpu.VMEM((EPW,), jnp.int32),
            pltpu.VMEM((EPW,), jnp.int32),
            pltpu.VMEM((NCH, CH), jnp.int32),
            pltpu.VMEM((BPW,), jnp.float32),
            pltpu.VMEM((CH,), jnp.float32),
            pltpu.VMEM_SHARED((NBINS,), jnp.float32),
            pltpu.SemaphoreType.DMA,
        ],
    )
    def k(dst_hbm, rel_hbm, out_hbm, dv, rv, key2d, zbuf, ones, binsh, sem):
        wid = _wid()
        sid = lax.axis_index("s")
        cid = lax.axis_index("c")
        base = wid * EPW
        pltpu.async_copy(dst_hbm.at[pl.ds(base, EPW)], dv, sem).wait()
        pltpu.async_copy(rel_hbm.at[pl.ds(base, EPW)], rv, sem).wait()

        z16 = jnp.zeros((16,), jnp.float32)
        o16 = jnp.full((16,), 1.0, jnp.float32)

        @pl.loop(0, CH // 16)
        def _(i):
            ones[pl.ds(i * 16, 16)] = o16

        @pl.loop(0, BPW // 16)
        def _(i):
            zbuf[pl.ds(i * 16, 16)] = z16

        @pl.loop(0, NCH)
        def _(c):
            @pl.loop(0, CH // 16)
            def _(j):
                kd = dv[pl.ds(c * CH + j * 16, 16)]
                kr = rv[pl.ds(c * CH + j * 16, 16)]
                key2d[c, pl.ds(j * 16, 16)] = kd * 16 + kr

        pltpu.async_copy(zbuf, binsh.at[pl.ds(sid * BPW, BPW)], sem).wait()
        plsc.subcore_barrier()

        @pl.loop(0, NCH)
        def _(c):
            pltpu.sync_copy(ones, binsh.at[key2d.at[c]], add=True)

        plsc.subcore_barrier()
        pltpu.sync_copy(binsh.at[pl.ds(sid * BPW, BPW)], zbuf)
        pltpu.async_copy(
            zbuf, out_hbm.at[pl.ds(cid * NBINS + sid * BPW, BPW)], sem).wait()

    return k(dst, rel)


@jax.jit
def _sc_agg(table, recflat, src, dst, rel):
    @functools.partial(
        pl.kernel,
        out_type=jax.ShapeDtypeStruct((NC, NPAD, D), jnp.float32),
        mesh=_mesh,
        scratch_types=[
            pltpu.VMEM((CH,), jnp.int32),
            pltpu.VMEM((CH,), jnp.int32),
            pltpu.VMEM((CH,), jnp.int32),
            pltpu.VMEM((2, CH), jnp.int32),
            pltpu.VMEM((2, CH), jnp.int32),
            pltpu.VMEM((2, CH), jnp.int32),
            pltpu.VMEM((2, CH, D), jnp.float32),
            pltpu.VMEM((2, CH), jnp.float32),
            pltpu.VMEM_SHARED((NPAD, D), jnp.float32),
            pltpu.SemaphoreType.DMA,
            pltpu.SemaphoreType.DMA,
            pltpu.SemaphoreType.DMA,
            pltpu.SemaphoreType.DMA,
            pltpu.SemaphoreType.DMA,
            pltpu.SemaphoreType.DMA,
            pltpu.SemaphoreType.DMA,
            pltpu.SemaphoreType.DMA,
            pltpu.SemaphoreType.DMA,
        ],
        compiler_params=_cp,
    )
    def k(tab_hbm, rec_hbm, src_hbm, dst_hbm, rel_hbm, out_hbm,
          bs, br, bd, bg, bl, dix, msg, normv, aggsh,
          si1, si2, si3, sg0, sg1, sn0, sn1, ss0, ss1):
        wid = _wid()
        sid = lax.axis_index("s")
        cid = lax.axis_index("c")
        base = wid * EPW
        sg = (sg0, sg1)
        sn = (sn0, sn1)
        ss = (ss0, ss1)

        z16 = jnp.zeros((16,), jnp.float32)

        @pl.loop(0, CH * (D // 16))
        def _(i):
            msg[0, i // (D // 16), pl.ds((i % (D // 16)) * 16, 16)] = z16

        for kblk in range(RPS // CH):
            pltpu.sync_copy(msg.at[0], aggsh.at[pl.ds(sid * RPS + kblk * CH, CH)])
        plsc.subcore_barrier()

        def idx_load(c):
            bc = base + c * CH
            pltpu.async_copy(src_hbm.at[pl.ds(bc, CH)], bs, si1)
            pltpu.async_copy(rel_hbm.at[pl.ds(bc, CH)], br, si2)
            pltpu.async_copy(dst_hbm.at[pl.ds(bc, CH)], bd, si3)

        def idx_wait_math(p):
            pltpu.make_async_copy(src_hbm.at[pl.ds(0, CH)], bs, si1).wait()
            pltpu.make_async_copy(rel_hbm.at[pl.ds(0, CH)], br, si2).wait()
            pltpu.make_async_copy(dst_hbm.at[pl.ds(0, CH)], bd, si3).wait()

            @pl.loop(0, CH // 16)
            def _(j):
                sl = pl.ds(j * 16, 16)
                d16 = bd[sl]
                r16 = br[sl]
                bg[p, sl] = r16 * N + bs[sl]
                bl[p, sl] = d16 * 16 + r16
                dix[p, sl] = d16

        def gathers(p):
            pltpu.async_copy(tab_hbm.at[bg.at[p]], msg.at[p], sg[p])
            pltpu.async_copy(rec_hbm.at[bl.at[p]], normv.at[p], sn[p])

        def wait_gathers(p):
            pltpu.make_async_copy(tab_hbm.at[bg.at[p]], msg.at[p], sg[p]).wait()
            pltpu.make_async_copy(rec_hbm.at[bl.at[p]], normv.at[p], sn[p]).wait()

        def scale(p):
            @pl.loop(0, CH)
            def _(e):
                ns = plsc.load_gather(
                    normv, [jnp.full((16,), p, jnp.int32),
                            jnp.full((16,), e, jnp.int32)])
                for kk in range(D // 16):
                    sl = pl.ds(kk * 16, 16)
                    msg[p, e, sl] = msg[p, e, sl] * ns

        def scatter(p):
            pltpu.async_copy(msg.at[p], aggsh.at[dix.at[p]], ss[p], add=True)

        def wait_scatter(p):
            pltpu.make_async_copy(msg.at[p], aggsh.at[dix.at[p]], ss[p]).wait()

        idx_load(0)
        idx_wait_math(0)
        gathers(0)
        idx_load(1)

        def body(c, p):
            q = 1 - p

            @pl.when(c >= 2)
            def _():
                wait_scatter(p)

            idx_wait_math(p)
            gathers(p)

            @pl.when(c < NCH - 1)
            def _():
                idx_load(c + 1)

            wait_gathers(q)
            scale(q)
            scatter(q)

        @pl.loop(0, (NCH - 1) // 2)
        def _(j):
            body(1 + 2 * j, 1)
            body(2 + 2 * j, 0)

        wait_gathers(0)
        scale(0)
        scatter(0)
        wait_scatter(1)
        wait_scatter(0)
        plsc.subcore_barrier()
        for kblk in range(RPS // CH):
            pltpu.sync_copy(aggsh.at[pl.ds(sid * RPS + kblk * CH, CH)], msg.at[0])
            pltpu.sync_copy(msg.at[0],
                            out_hbm.at[cid, pl.ds(sid * RPS + kblk * CH, CH)])

    return k(table, recflat, src, dst, rel)


NBUF = 4


@jax.jit
def _sc_gab(A, B, src, dst):
    @functools.partial(
        pl.kernel,
        out_type=[jax.ShapeDtypeStruct((E, D), jnp.float32),
                  jax.ShapeDtypeStruct((E, D), jnp.float32)],
        mesh=_mesh,
        scratch_types=[
            pltpu.VMEM((EPW,), jnp.int32),
            pltpu.VMEM((EPW,), jnp.int32),
            pltpu.VMEM((NBUF, CH, D), jnp.float32),
            pltpu.VMEM((NBUF, CH, D), jnp.float32),
        ] + [pltpu.SemaphoreType.DMA] * (4 * NBUF),
    )
    def k(a_hbm, b_hbm, src_hbm, dst_hbm, ga_hbm, gb_hbm,
          sv, dv, bufa, bufb, *sems):
        wid = _wid()
        base = wid * EPW
        sga = sems[0:NBUF]
        sgb = sems[NBUF:2 * NBUF]
        swa = sems[2 * NBUF:3 * NBUF]
        swb = sems[3 * NBUF:4 * NBUF]
        pltpu.async_copy(src_hbm.at[pl.ds(base, EPW)], sv, sga[0]).wait()
        pltpu.async_copy(dst_hbm.at[pl.ds(base, EPW)], dv, sga[0]).wait()

        def gath(c, p):
            pltpu.async_copy(a_hbm.at[sv.at[pl.ds(c * CH, CH)]],
                             bufa.at[p], sga[p])
            pltpu.async_copy(b_hbm.at[dv.at[pl.ds(c * CH, CH)]],
                             bufb.at[p], sgb[p])

        def wait_gath(p):
            pltpu.make_async_copy(a_hbm.at[sv.at[pl.ds(0, CH)]],
                                  bufa.at[p], sga[p]).wait()
            pltpu.make_async_copy(b_hbm.at[dv.at[pl.ds(0, CH)]],
                                  bufb.at[p], sgb[p]).wait()

        def wr(c, p):
            pltpu.async_copy(bufa.at[p], ga_hbm.at[pl.ds(base + c * CH, CH)],
                             swa[p])
            pltpu.async_copy(bufb.at[p], gb_hbm.at[pl.ds(base + c * CH, CH)],
                             swb[p])

        def wait_wr(p):
            pltpu.make_async_copy(bufa.at[p], ga_hbm.at[pl.ds(0, CH)],
                                  swa[p]).wait()
            pltpu.make_async_copy(bufb.at[p], gb_hbm.at[pl.ds(0, CH)],
                                  swb[p]).wait()

        for c0 in range(NBUF - 1):
            gath(c0, c0)

        def body(c, p):
            wait_gath(p)
            wr(c, p)

            @pl.when(c >= 1)
            def _():
                wait_wr((p + NBUF - 1) % NBUF)

            @pl.when(c + NBUF - 1 < NCH)
            def _():
                gath(c + NBUF - 1, (p + NBUF - 1) % NBUF)

        @pl.loop(0, (NCH - 1) // NBUF)
        def _(j):
            for pp in range(NBUF):
                body(NBUF * j + pp, pp)

        body(NCH - 1, (NCH - 1) % NBUF)
        wait_wr((NCH - 1) % NBUF)

    return k(A, B, src, dst)


def _recip_body(b_ref, o_ref):
    o_ref[...] = 1.0 / jnp.maximum(b_ref[0] + b_ref[1], 1.0)


@jax.jit
def _tc_recip(bins):
    nr = NBINS // 128
    return pl.pallas_call(
        _recip_body,
        out_shape=jax.ShapeDtypeStruct((nr, 128), jnp.float32),
    )(bins.reshape(NC, nr, 128))


def _proj_body(x_ref, w_ref, o_ref):
    o_ref[0] = jnp.dot(x_ref[...].astype(jnp.bfloat16),
                       w_ref[0].astype(jnp.bfloat16),
                       preferred_element_type=jnp.float32)


@jax.jit
def _tc_proj(x, wcat):
    bn = 1000
    return pl.pallas_call(
        _proj_body,
        grid=(N // bn, R + 1),
        in_specs=[
            pl.BlockSpec((bn, D), lambda i, r: (i, 0)),
            pl.BlockSpec((1, D, D), lambda i, r: (r, 0, 0)),
        ],
        out_specs=pl.BlockSpec((1, bn, D), lambda i, r: (r, i, 0)),
        out_shape=jax.ShapeDtypeStruct((R + 1, N, D), jnp.float32),
    )(x, wcat)


def _layer_body(a_ref, xr_ref, b_ref, w_ref, o_ref):
    h = jnp.maximum(a_ref[0] + a_ref[1] + xr_ref[0] + b_ref[...], 0.0)
    o_ref[0] = jnp.dot(h.astype(jnp.bfloat16), w_ref[0].astype(jnp.bfloat16),
                       preferred_element_type=jnp.float32)


@jax.jit
def _tc_layer(agg, proj_prev, b, wcat):
    bn = 1000
    return pl.pallas_call(
        _layer_body,
        grid=(N // bn, R + 1),
        in_specs=[
            pl.BlockSpec((NC, bn, D), lambda i, r: (0, i, 0)),
            pl.BlockSpec((1, bn, D), lambda i, r: (R, i, 0)),
            pl.BlockSpec((1, D), lambda i, r: (0, 0)),
            pl.BlockSpec((1, D, D), lambda i, r: (r, 0, 0)),
        ],
        out_specs=pl.BlockSpec((1, bn, D), lambda i, r: (r, i, 0)),
        out_shape=jax.ShapeDtypeStruct((R + 1, N, D), jnp.float32),
    )(agg, proj_prev, b, wcat)


def _head_body(a_ref, xr_ref, b_ref, wa_ref, wb_ref, bm_ref, oa_ref, ob_ref):
    h = jnp.maximum(a_ref[0] + a_ref[1] + xr_ref[0] + b_ref[...], 0.0)
    oa_ref[...] = jnp.dot(h, wa_ref[...],
                          preferred_element_type=jnp.float32) + bm_ref[...]
    ob_ref[...] = jnp.dot(h, wb_ref[...], preferred_element_type=jnp.float32)


@jax.jit
def _tc_head(agg, proj_prev, b, wa, wb, bm):
    bn = 1000
    return pl.pallas_call(
        _head_body,
        grid=(N // bn,),
        in_specs=[
            pl.BlockSpec((NC, bn, D), lambda i: (0, i, 0)),
            pl.BlockSpec((1, bn, D), lambda i: (R, i, 0)),
            pl.BlockSpec((1, D), lambda i: (0, 0)),
            pl.BlockSpec((D, D), lambda i: (0, 0)),
            pl.BlockSpec((D, D), lambda i: (0, 0)),
            pl.BlockSpec((1, D), lambda i: (0, 0)),
        ],
        out_specs=[pl.BlockSpec((bn, D), lambda i: (i, 0)),
                   pl.BlockSpec((bn, D), lambda i: (i, 0))],
        out_shape=[jax.ShapeDtypeStruct((N, D), jnp.float32),
                   jax.ShapeDtypeStruct((N, D), jnp.float32)],
    )(agg, proj_prev, b, wa, wb, bm)


def _mlp_body(e_ref, m_ref, ga_ref, gb_ref, wt_ref, w2_ref, b2_ref, o_ref):
    mtxt = e_ref[...] * m_ref[...]
    z = jnp.maximum(
        jnp.dot(mtxt.astype(jnp.bfloat16), wt_ref[...].astype(jnp.bfloat16),
                preferred_element_type=jnp.float32)
        + ga_ref[...] + gb_ref[...], 0.0)
    o_ref[...] = jnp.dot(z, w2_ref[...],
                         preferred_element_type=jnp.float32) + b2_ref[...]


@jax.jit
def _tc_mlp(etext, mask2d, ga, gb, wt, w2, b2):
    be = 2000
    return pl.pallas_call(
        _mlp_body,
        grid=(E // be,),
        in_specs=[
            pl.BlockSpec((be, D), lambda i: (i, 0)),
            pl.BlockSpec((be, 1), lambda i: (i, 0)),
            pl.BlockSpec((be, D), lambda i: (i, 0)),
            pl.BlockSpec((be, D), lambda i: (i, 0)),
            pl.BlockSpec((D, D), lambda i: (0, 0)),
            pl.BlockSpec((D, 1), lambda i: (0, 0)),
            pl.BlockSpec((1, 1), lambda i: (0, 0)),
        ],
        out_specs=pl.BlockSpec((be, 1), lambda i: (i, 0)),
        out_shape=jax.ShapeDtypeStruct((E, 1), jnp.float32),
    )(etext, mask2d, ga, gb, wt, w2, b2)


def kernel(full_edge_index, full_edge_type, full_edge_text_emb, sub_edge_idx,
           m_logits, node_emb, W1, root1, b1, W2, root2, b2, Wm1, bm1, Wm2, bm2):
    src = full_edge_index[0]
    dst = full_edge_index[1]
    rel = full_edge_type

    mask_full = _sc_mask(sub_edge_idx, m_logits)
    bins = _sc_hist(dst, rel)
    recip = _tc_recip(bins).reshape(NBINS)

    w1cat = jnp.concatenate([W1, root1[None]], axis=0)
    proj1 = _tc_proj(node_emb, w1cat)
    agg1 = _sc_agg(proj1.reshape((R + 1) * N, D), recip, src, dst, rel)

    w2cat = jnp.concatenate([W2, root2[None]], axis=0)
    proj2 = _tc_layer(agg1, proj1, b1.reshape(1, D), w2cat)
    agg2 = _sc_agg(proj2.reshape((R + 1) * N, D), recip, src, dst, rel)

    A, B = _tc_head(agg2, proj2, b2.reshape(1, D),
                    Wm1[:D], Wm1[D:2 * D], bm1.reshape(1, D))
    gA, gB = _sc_gab(A, B, src, dst)

    logits = _tc_mlp(full_edge_text_emb, mask_full.reshape(E, 1), gA, gB,
                     Wm1[2 * D:], Wm2, bm2.reshape(1, 1))
    return (logits.reshape(E), mask_full)


# speedup vs baseline: 1.0089x; 1.0089x over previous
import dataclasses
import functools

import jax
import jax.numpy as jnp
from jax import lax
from jax.experimental import pallas as pl
from jax.experimental.pallas import tpu as pltpu
from jax.experimental.pallas import tpu_sc as plsc

N = 10000
R = 16
E = 320000
S = 4096
D = 128

NC = 2
NS = 16
NW = NC * NS
EPW = E // NW
CH = 80
NCH = EPW // CH
NBINS = 160256
BPW = NBINS // NS
NPAD = 10240
RPS = NPAD // NS

_mesh = plsc.VectorSubcoreMesh(core_axis_name="c", subcore_axis_name="s")

_cp = pltpu.CompilerParams()
if "needs_layout_passes" in pltpu.CompilerParams.__dataclass_fields__:
    _cp = dataclasses.replace(_cp, needs_layout_passes=False)


def _wid():
    return lax.axis_index("s") * NC + lax.axis_index("c")


@jax.jit
def _sc_mask(sub_idx, m_logits):
    @functools.partial(
        pl.kernel,
        out_type=jax.ShapeDtypeStruct((E,), jnp.float32),
        mesh=_mesh,
        scratch_types=[
            pltpu.VMEM((S,), jnp.int32),
            pltpu.VMEM((S,), jnp.float32),
            pltpu.VMEM((EPW,), jnp.float32),
            pltpu.SemaphoreType.DMA,
        ],
        compiler_params=_cp,
    )
    def k(idx_hbm, logit_hbm, out_hbm, idx_v, logit_v, buf, sem):
        wid = _wid()
        base = wid * EPW
        pltpu.async_copy(idx_hbm, idx_v, sem).wait()
        pltpu.async_copy(logit_hbm, logit_v, sem).wait()

        ones = jnp.full((16,), 1.0, jnp.float32)

        @pl.loop(0, EPW // 16)
        def _(i):
            buf[pl.ds(i * 16, 16)] = ones

        @pl.loop(0, S // 16)
        def _(j):
            iv = idx_v[pl.ds(j * 16, 16)]
            lv = logit_v[pl.ds(j * 16, 16)]
            val = 1.0 / (1.0 + jnp.exp(-lv))
            m = (iv >= base) & (iv < base + EPW)
            il = jnp.minimum(jnp.maximum(iv - base, 0), EPW - 1)
            plsc.store_scatter(buf, [il], val, mask=m)

        pltpu.async_copy(buf, out_hbm.at[pl.ds(base, EPW)], sem).wait()

    return k(sub_idx, m_logits)


@jax.jit
def _sc_hist(dst, rel):
    @functools.partial(
        pl.kernel,
        out_type=jax.ShapeDtypeStruct((NC * NBINS,), jnp.float32),
        mesh=_mesh,
        scratch_types=[
            pltpu.VMEM((EPW,), jnp.int32),
            pltpu.VMEM((EPW,), jnp.int32),
            pltpu.VMEM((NCH, CH), jnp.int32),
            pltpu.VMEM((BPW,), jnp.float32),
            pltpu.VMEM((CH,), jnp.float32),
            pltpu.VMEM_SHARED((NBINS,), jnp.float32),
            pltpu.SemaphoreType.DMA,
        ],
    )
    def k(dst_hbm, rel_hbm, out_hbm, dv, rv, key2d, zbuf, ones, binsh, sem):
        wid = _wid()
        sid = lax.axis_index("s")
        cid = lax.axis_index("c")
        base = wid * EPW
        pltpu.async_copy(dst_hbm.at[pl.ds(base, EPW)], dv, sem).wait()
        pltpu.async_copy(rel_hbm.at[pl.ds(base, EPW)], rv, sem).wait()

        z16 = jnp.zeros((16,), jnp.float32)
        o16 = jnp.full((16,), 1.0, jnp.float32)

        @pl.loop(0, CH // 16)
        def _(i):
            ones[pl.ds(i * 16, 16)] = o16

        @pl.loop(0, BPW // 16)
        def _(i):
            zbuf[pl.ds(i * 16, 16)] = z16

        @pl.loop(0, NCH)
        def _(c):
            @pl.loop(0, CH // 16)
            def _(j):
                kd = dv[pl.ds(c * CH + j * 16, 16)]
                kr = rv[pl.ds(c * CH + j * 16, 16)]
                key2d[c, pl.ds(j * 16, 16)] = kd * 16 + kr

        pltpu.async_copy(zbuf, binsh.at[pl.ds(sid * BPW, BPW)], sem).wait()
        plsc.subcore_barrier()

        @pl.loop(0, NCH)
        def _(c):
            pltpu.sync_copy(ones, binsh.at[key2d.at[c]], add=True)

        plsc.subcore_barrier()
        pltpu.sync_copy(binsh.at[pl.ds(sid * BPW, BPW)], zbuf)
        pltpu.async_copy(
            zbuf, out_hbm.at[pl.ds(cid * NBINS + sid * BPW, BPW)], sem).wait()

    return k(dst, rel)


@jax.jit
def _sc_agg(table, recflat, src, dst, rel):
    @functools.partial(
        pl.kernel,
        out_type=jax.ShapeDtypeStruct((NC, NPAD, D), jnp.float32),
        mesh=_mesh,
        scratch_types=[
            pltpu.VMEM((CH,), jnp.int32),
            pltpu.VMEM((CH,), jnp.int32),
            pltpu.VMEM((CH,), jnp.int32),
            pltpu.VMEM((2, CH), jnp.int32),
            pltpu.VMEM((2, CH), jnp.int32),
            pltpu.VMEM((2, CH), jnp.int32),
            pltpu.VMEM((2, CH, D), jnp.float32),
            pltpu.VMEM((2, CH), jnp.float32),
            pltpu.VMEM_SHARED((NPAD, D), jnp.float32),
            pltpu.SemaphoreType.DMA,
            pltpu.SemaphoreType.DMA,
            pltpu.SemaphoreType.DMA,
            pltpu.SemaphoreType.DMA,
            pltpu.SemaphoreType.DMA,
            pltpu.SemaphoreType.DMA,
            pltpu.SemaphoreType.DMA,
            pltpu.SemaphoreType.DMA,
            pltpu.SemaphoreType.DMA,
        ],
        compiler_params=_cp,
    )
    def k(tab_hbm, rec_hbm, src_hbm, dst_hbm, rel_hbm, out_hbm,
          bs, br, bd, bg, bl, dix, msg, normv, aggsh,
          si1, si2, si3, sg0, sg1, sn0, sn1, ss0, ss1):
        wid = _wid()
        sid = lax.axis_index("s")
        cid = lax.axis_index("c")
        base = wid * EPW
        sg = (sg0, sg1)
        sn = (sn0, sn1)
        ss = (ss0, ss1)

        z16 = jnp.zeros((16,), jnp.float32)

        @pl.loop(0, CH * (D // 16))
        def _(i):
            msg[0, i // (D // 16), pl.ds((i % (D // 16)) * 16, 16)] = z16

        for kblk in range(RPS // CH):
            pltpu.sync_copy(msg.at[0], aggsh.at[pl.ds(sid * RPS + kblk * CH, CH)])
        plsc.subcore_barrier()

        def idx_load(c):
            bc = base + c * CH
            pltpu.async_copy(src_hbm.at[pl.ds(bc, CH)], bs, si1)
            pltpu.async_copy(rel_hbm.at[pl.ds(bc, CH)], br, si2)
            pltpu.async_copy(dst_hbm.at[pl.ds(bc, CH)], bd, si3)

        def idx_wait_math(p):
            pltpu.make_async_copy(src_hbm.at[pl.ds(0, CH)], bs, si1).wait()
            pltpu.make_async_copy(rel_hbm.at[pl.ds(0, CH)], br, si2).wait()
            pltpu.make_async_copy(dst_hbm.at[pl.ds(0, CH)], bd, si3).wait()

            @pl.loop(0, CH // 16)
            def _(j):
                sl = pl.ds(j * 16, 16)
                d16 = bd[sl]
                r16 = br[sl]
                bg[p, sl] = r16 * N + bs[sl]
                bl[p, sl] = d16 * 16 + r16
                dix[p, sl] = d16

        def gathers(p):
            pltpu.async_copy(tab_hbm.at[bg.at[p]], msg.at[p], sg[p])
            pltpu.async_copy(rec_hbm.at[bl.at[p]], normv.at[p], sn[p])

        def wait_gathers(p):
            pltpu.make_async_copy(tab_hbm.at[bg.at[p]], msg.at[p], sg[p]).wait()
            pltpu.make_async_copy(rec_hbm.at[bl.at[p]], normv.at[p], sn[p]).wait()

        def scale(p):
            @pl.loop(0, CH)
            def _(e):
                ns = plsc.load_gather(
                    normv, [jnp.full((16,), p, jnp.int32),
                            jnp.full((16,), e, jnp.int32)])
                for kk in range(D // 16):
                    sl = pl.ds(kk * 16, 16)
                    msg[p, e, sl] = msg[p, e, sl] * ns

        def scatter(p):
            pltpu.async_copy(msg.at[p], aggsh.at[dix.at[p]], ss[p], add=True)

        def wait_scatter(p):
            pltpu.make_async_copy(msg.at[p], aggsh.at[dix.at[p]], ss[p]).wait()

        idx_load(0)
        idx_wait_math(0)
        gathers(0)
        idx_load(1)

        def body(c, p):
            q = 1 - p

            @pl.when(c >= 2)
            def _():
                wait_scatter(p)

            idx_wait_math(p)
            gathers(p)

            @pl.when(c < NCH - 1)
            def _():
                idx_load(c + 1)

            wait_gathers(q)
            scale(q)
            scatter(q)

        @pl.loop(0, (NCH - 1) // 2)
        def _(j):
            body(1 + 2 * j, 1)
            body(2 + 2 * j, 0)

        wait_gathers(0)
        scale(0)
        scatter(0)
        wait_scatter(1)
        wait_scatter(0)
        plsc.subcore_barrier()
        for kblk in range(RPS // CH):
            pltpu.sync_copy(aggsh.at[pl.ds(sid * RPS + kblk * CH, CH)], msg.at[0])
            pltpu.sync_copy(msg.at[0],
                            out_hbm.at[cid, pl.ds(sid * RPS + kblk * CH, CH)])

    return k(table, recflat, src, dst, rel)


NBUF = 4


@jax.jit
def _sc_gab(A, B, src, dst):
    @functools.partial(
        pl.kernel,
        out_type=[jax.ShapeDtypeStruct((E, D), jnp.float32),
                  jax.ShapeDtypeStruct((E, D), jnp.float32)],
        mesh=_mesh,
        scratch_types=[
            pltpu.VMEM((EPW,), jnp.int32),
            pltpu.VMEM((EPW,), jnp.int32),
            pltpu.VMEM((NBUF, CH, D), jnp.float32),
            pltpu.VMEM((NBUF, CH, D), jnp.float32),
        ] + [pltpu.SemaphoreType.DMA] * (4 * NBUF),
    )
    def k(a_hbm, b_hbm, src_hbm, dst_hbm, ga_hbm, gb_hbm,
          sv, dv, bufa, bufb, *sems):
        wid = _wid()
        base = wid * EPW
        sga = sems[0:NBUF]
        sgb = sems[NBUF:2 * NBUF]
        swa = sems[2 * NBUF:3 * NBUF]
        swb = sems[3 * NBUF:4 * NBUF]
        pltpu.async_copy(src_hbm.at[pl.ds(base, EPW)], sv, sga[0]).wait()
        pltpu.async_copy(dst_hbm.at[pl.ds(base, EPW)], dv, sga[0]).wait()

        def gath(c, p):
            pltpu.async_copy(a_hbm.at[sv.at[pl.ds(c * CH, CH)]],
                             bufa.at[p], sga[p])
            pltpu.async_copy(b_hbm.at[dv.at[pl.ds(c * CH, CH)]],
                             bufb.at[p], sgb[p])

        def wait_gath(p):
            pltpu.make_async_copy(a_hbm.at[sv.at[pl.ds(0, CH)]],
                                  bufa.at[p], sga[p]).wait()
            pltpu.make_async_copy(b_hbm.at[dv.at[pl.ds(0, CH)]],
                                  bufb.at[p], sgb[p]).wait()

        def wr(c, p):
            pltpu.async_copy(bufa.at[p], ga_hbm.at[pl.ds(base + c * CH, CH)],
                             swa[p])
            pltpu.async_copy(bufb.at[p], gb_hbm.at[pl.ds(base + c * CH, CH)],
                             swb[p])

        def wait_wr(p):
            pltpu.make_async_copy(bufa.at[p], ga_hbm.at[pl.ds(0, CH)],
                                  swa[p]).wait()
            pltpu.make_async_copy(bufb.at[p], gb_hbm.at[pl.ds(0, CH)],
                                  swb[p]).wait()

        for c0 in range(NBUF - 1):
            gath(c0, c0)

        def body(c, p):
            wait_gath(p)
            wr(c, p)

            @pl.when(c >= 1)
            def _():
                wait_wr((p + NBUF - 1) % NBUF)

            @pl.when(c + NBUF - 1 < NCH)
            def _():
                gath(c + NBUF - 1, (p + NBUF - 1) % NBUF)

        @pl.loop(0, (NCH - 1) // NBUF)
        def _(j):
            for pp in range(NBUF):
                body(NBUF * j + pp, pp)

        body(NCH - 1, (NCH - 1) % NBUF)
        wait_wr((NCH - 1) % NBUF)

    return k(A, B, src, dst)


def _recip_body(b_ref, o_ref):
    o_ref[...] = 1.0 / jnp.maximum(b_ref[0] + b_ref[1], 1.0)


@jax.jit
def _tc_recip(bins):
    nr = NBINS // 128
    return pl.pallas_call(
        _recip_body,
        out_shape=jax.ShapeDtypeStruct((nr, 128), jnp.float32),
    )(bins.reshape(NC, nr, 128))


def _proj_body(x_ref, w_ref, o_ref):
    o_ref[0] = jnp.dot(x_ref[...], w_ref[0],
                       preferred_element_type=jnp.float32)


@jax.jit
def _tc_proj(x, wcat):
    bn = 1000
    return pl.pallas_call(
        _proj_body,
        grid=(N // bn, R + 1),
        in_specs=[
            pl.BlockSpec((bn, D), lambda i, r: (i, 0)),
            pl.BlockSpec((1, D, D), lambda i, r: (r, 0, 0)),
        ],
        out_specs=pl.BlockSpec((1, bn, D), lambda i, r: (r, i, 0)),
        out_shape=jax.ShapeDtypeStruct((R + 1, N, D), jnp.float32),
    )(x, wcat)


def _layer_body(a_ref, xr_ref, b_ref, w_ref, o_ref):
    h = jnp.maximum(a_ref[0] + a_ref[1] + xr_ref[0] + b_ref[...], 0.0)
    o_ref[0] = jnp.dot(h, w_ref[0], preferred_element_type=jnp.float32)


@jax.jit
def _tc_layer(agg, proj_prev, b, wcat):
    bn = 1000
    return pl.pallas_call(
        _layer_body,
        grid=(N // bn, R + 1),
        in_specs=[
            pl.BlockSpec((NC, bn, D), lambda i, r: (0, i, 0)),
            pl.BlockSpec((1, bn, D), lambda i, r: (R, i, 0)),
            pl.BlockSpec((1, D), lambda i, r: (0, 0)),
            pl.BlockSpec((1, D, D), lambda i, r: (r, 0, 0)),
        ],
        out_specs=pl.BlockSpec((1, bn, D), lambda i, r: (r, i, 0)),
        out_shape=jax.ShapeDtypeStruct((R + 1, N, D), jnp.float32),
    )(agg, proj_prev, b, wcat)


def _head_body(a_ref, xr_ref, b_ref, wa_ref, wb_ref, bm_ref, oa_ref, ob_ref):
    h = jnp.maximum(a_ref[0] + a_ref[1] + xr_ref[0] + b_ref[...], 0.0)
    oa_ref[...] = jnp.dot(h, wa_ref[...],
                          preferred_element_type=jnp.float32) + bm_ref[...]
    ob_ref[...] = jnp.dot(h, wb_ref[...], preferred_element_type=jnp.float32)


@jax.jit
def _tc_head(agg, proj_prev, b, wa, wb, bm):
    bn = 1000
    return pl.pallas_call(
        _head_body,
        grid=(N // bn,),
        in_specs=[
            pl.BlockSpec((NC, bn, D), lambda i: (0, i, 0)),
            pl.BlockSpec((1, bn, D), lambda i: (R, i, 0)),
            pl.BlockSpec((1, D), lambda i: (0, 0)),
            pl.BlockSpec((D, D), lambda i: (0, 0)),
            pl.BlockSpec((D, D), lambda i: (0, 0)),
            pl.BlockSpec((1, D), lambda i: (0, 0)),
        ],
        out_specs=[pl.BlockSpec((bn, D), lambda i: (i, 0)),
                   pl.BlockSpec((bn, D), lambda i: (i, 0))],
        out_shape=[jax.ShapeDtypeStruct((N, D), jnp.float32),
                   jax.ShapeDtypeStruct((N, D), jnp.float32)],
    )(agg, proj_prev, b, wa, wb, bm)


def _mlp_body(e_ref, m_ref, ga_ref, gb_ref, wt_ref, w2_ref, b2_ref, o_ref):
    mtxt = e_ref[...] * m_ref[...]
    z = jnp.maximum(
        jnp.dot(mtxt, wt_ref[...], preferred_element_type=jnp.float32)
        + ga_ref[...] + gb_ref[...], 0.0)
    o_ref[...] = jnp.dot(z, w2_ref[...],
                         preferred_element_type=jnp.float32) + b2_ref[...]


@jax.jit
def _tc_mlp(etext, mask2d, ga, gb, wt, w2, b2):
    be = 2000
    return pl.pallas_call(
        _mlp_body,
        grid=(E // be,),
        in_specs=[
            pl.BlockSpec((be, D), lambda i: (i, 0)),
            pl.BlockSpec((be, 1), lambda i: (i, 0)),
            pl.BlockSpec((be, D), lambda i: (i, 0)),
            pl.BlockSpec((be, D), lambda i: (i, 0)),
            pl.BlockSpec((D, D), lambda i: (0, 0)),
            pl.BlockSpec((D, 1), lambda i: (0, 0)),
            pl.BlockSpec((1, 1), lambda i: (0, 0)),
        ],
        out_specs=pl.BlockSpec((be, 1), lambda i: (i, 0)),
        out_shape=jax.ShapeDtypeStruct((E, 1), jnp.float32),
    )(etext, mask2d, ga, gb, wt, w2, b2)


def kernel(full_edge_index, full_edge_type, full_edge_text_emb, sub_edge_idx,
           m_logits, node_emb, W1, root1, b1, W2, root2, b2, Wm1, bm1, Wm2, bm2):
    src = full_edge_index[0]
    dst = full_edge_index[1]
    rel = full_edge_type

    mask_full = _sc_mask(sub_edge_idx, m_logits)
    bins = _sc_hist(dst, rel)
    recip = _tc_recip(bins).reshape(NBINS)

    w1cat = jnp.concatenate([W1, root1[None]], axis=0)
    proj1 = _tc_proj(node_emb, w1cat)
    agg1 = _sc_agg(proj1.reshape((R + 1) * N, D), recip, src, dst, rel)

    w2cat = jnp.concatenate([W2, root2[None]], axis=0)
    proj2 = _tc_layer(agg1, proj1, b1.reshape(1, D), w2cat)
    agg2 = _sc_agg(proj2.reshape((R + 1) * N, D), recip, src, dst, rel)

    A, B = _tc_head(agg2, proj2, b2.reshape(1, D),
                    Wm1[:D], Wm1[D:2 * D], bm1.reshape(1, D))
    gA, gB = _sc_gab(A, B, src, dst)

    logits = _tc_mlp(full_edge_text_emb, mask_full.reshape(E, 1), gA, gB,
                     Wm1[2 * D:], Wm2, bm2.reshape(1, 1))
    return (logits.reshape(E), mask_full)
